# baseline (speedup 1.0000x reference)
import functools

import jax
import jax.numpy as jnp
from jax import lax
from jax.experimental import pallas as pl
from jax.experimental.pallas import tpu as pltpu
from jax.experimental.pallas import tpu_sc as plsc

_NC = 2
_NS = 16
_NW = _NC * _NS
_L = 16

_CS = 8


def _sc_body(seq, d, batch, x_ref, pos_ref, out_ref, pos_v, x_v, sems):
    wid = lax.axis_index("s") * _NC + lax.axis_index("c")
    sp = seq // _NW
    base = wid * sp
    n_chunks = sp // _CS

    def in_copies(ci, slot):
        row = base + ci * _CS
        hp = pltpu.make_async_copy(
            pos_ref.at[pl.ds(row, _CS), :], pos_v.at[slot], sems.at[slot, 0]
        )
        hx = pltpu.make_async_copy(
            x_ref.at[:, pl.ds(row, _CS), :], x_v.at[slot], sems.at[slot, 1]
        )
        return hp, hx

    def out_copy(ci, slot):
        row = base + ci * _CS
        return pltpu.make_async_copy(
            x_v.at[slot], out_ref.at[:, pl.ds(row, _CS), :], sems.at[slot, 2]
        )

    def issue_in(ci, slot):
        hp, hx = in_copies(ci, slot)
        hp.start()
        hx.start()

    issue_in(0, 0)

    def chunk_step(ci, carry):
        slot = lax.rem(ci, 2)
        nxt = 1 - slot

        @pl.when(ci + 1 < n_chunks)
        def _prefetch():
            @pl.when(ci >= 1)
            def _drain_prev_out():
                out_copy(ci - 1, nxt).wait()

            issue_in(ci + 1, nxt)

        hp, hx = in_copies(ci, slot)
        hp.wait()
        hx.wait()

        def add_group(c, _):
            for r in range(_CS):
                pv = pos_v[slot, r, pl.ds(c, _L)]
                for b in range(batch):
                    x_v[slot, b, r, pl.ds(c, _L)] = (
                        x_v[slot, b, r, pl.ds(c, _L)] + pv
                    )
            return ()

        plsc.parallel_loop(0, d, _L, unroll=8, carry=())(add_group)
        out_copy(ci, slot).start()
        return carry

    lax.fori_loop(0, n_chunks, chunk_step, 0)

    out_copy(n_chunks - 2, (n_chunks - 2) % 2).wait()
    out_copy(n_chunks - 1, (n_chunks - 1) % 2).wait()


def kernel(x, pos_embedding):
    batch, seq, d = x.shape
    mesh = plsc.VectorSubcoreMesh(core_axis_name="c", subcore_axis_name="s")
    k = pl.kernel(
        functools.partial(_sc_body, seq, d, batch),
        out_type=jax.ShapeDtypeStruct((batch, seq, d), x.dtype),
        mesh=mesh,
        scratch_types=[
            pltpu.VMEM((2, _CS, d), jnp.float32),
            pltpu.VMEM((2, batch, _CS, d), jnp.float32),
            pltpu.SemaphoreType.DMA((2, 3)),
        ],
    )
    return k(x, pos_embedding)


# speedup vs baseline: 1.0052x; 1.0052x over previous
import functools

import jax
import jax.numpy as jnp
from jax import lax
from jax.experimental import pallas as pl
from jax.experimental.pallas import tpu as pltpu
from jax.experimental.pallas import tpu_sc as plsc

_NC = 2
_NS = 16
_NW = _NC * _NS
_L = 16

_CS = 8


def _sc_body(seq, d, batch, x_ref, pos_ref, out_ref, pos_v, x_v, sems):
    wid = lax.axis_index("s") * _NC + lax.axis_index("c")
    sp = seq // _NW
    base = wid * sp
    n_chunks = sp // _CS

    def in_copies(ci, slot):
        row = base + ci * _CS
        hp = pltpu.make_async_copy(
            pos_ref.at[pl.ds(row, _CS), :], pos_v.at[slot], sems.at[slot, 0]
        )
        hx = pltpu.make_async_copy(
            x_ref.at[:, pl.ds(row, _CS), :], x_v.at[slot], sems.at[slot, 1]
        )
        return hp, hx

    def out_copy(ci, slot):
        row = base + ci * _CS
        return pltpu.make_async_copy(
            x_v.at[slot], out_ref.at[:, pl.ds(row, _CS), :], sems.at[slot, 2]
        )

    def issue_in(ci, slot):
        hp, hx = in_copies(ci, slot)
        hp.start()
        hx.start()

    issue_in(0, 0)

    def chunk_step(ci, carry):
        slot = lax.rem(ci, 2)
        nxt = 1 - slot

        @pl.when(ci + 1 < n_chunks)
        def _prefetch():
            @pl.when(ci >= 1)
            def _drain_prev_out():
                out_copy(ci - 1, nxt).wait()

            issue_in(ci + 1, nxt)

        hp, hx = in_copies(ci, slot)
        hp.wait()
        hx.wait()

        def add_group(c, _):
            for r in range(_CS):
                pv = pos_v[slot, r, pl.ds(c, _L)]
                for b in range(batch):
                    x_v[slot, b, r, pl.ds(c, _L)] = (
                        x_v[slot, b, r, pl.ds(c, _L)] + pv
                    )
            return ()

        plsc.parallel_loop(0, d, _L, unroll=4, carry=())(add_group)
        out_copy(ci, slot).start()
        return carry

    lax.fori_loop(0, n_chunks, chunk_step, 0)

    out_copy(n_chunks - 2, (n_chunks - 2) % 2).wait()
    out_copy(n_chunks - 1, (n_chunks - 1) % 2).wait()


def kernel(x, pos_embedding):
    batch, seq, d = x.shape
    mesh = plsc.VectorSubcoreMesh(core_axis_name="c", subcore_axis_name="s")
    k = pl.kernel(
        functools.partial(_sc_body, seq, d, batch),
        out_type=jax.ShapeDtypeStruct((batch, seq, d), x.dtype),
        mesh=mesh,
        scratch_types=[
            pltpu.VMEM((2, _CS, d), jnp.float32),
            pltpu.VMEM((2, batch, _CS, d), jnp.float32),
            pltpu.SemaphoreType.DMA((2, 3)),
        ],
    )
    return k(x, pos_embedding)
